# baseline (speedup 1.0000x reference)
import jax
import jax.numpy as jnp
from jax import lax
from jax.experimental import pallas as pl
from jax.experimental.pallas import tpu as pltpu
from jax.experimental.pallas import tpu_sc as plsc

BATCH = 16384
HIDDEN = 64
NC = 2
NS = 16
NW = NC * NS
B_PER_W = BATCH // NW


def _sc_gather(tidx_hbm, tbl_hbm, out_hbm, idx_v, rows_v, gsem, wsem):
    wid = lax.axis_index("s") * NC + lax.axis_index("c")
    base = wid * B_PER_W
    pltpu.sync_copy(tidx_hbm.at[wid], idx_v)
    lane_iota = lax.iota(jnp.int32, 16)

    def fire_group(g, _):
        vec = idx_v[pl.ds(g * 16, 16)]
        for lane in range(16):
            s = jnp.sum(jnp.where(lane_iota == lane, vec, 0))
            pltpu.make_async_copy(
                tbl_hbm.at[pl.ds(s, 1), :],
                rows_v.at[pl.ds(g * 16 + lane, 1), :],
                gsem,
            ).start()
        return _

    lax.fori_loop(0, B_PER_W // 16, fire_group, None)
    pltpu.make_async_copy(
        tbl_hbm.at[pl.ds(0, B_PER_W), :], rows_v, gsem).wait()
    pltpu.sync_copy(rows_v, out_hbm.at[pl.ds(base, B_PER_W)])


@jax.jit
def _gather_rows(class_labels, table):
    tidx = class_labels.astype(jnp.int32).reshape(NW, B_PER_W)
    mesh = plsc.VectorSubcoreMesh(core_axis_name="c", subcore_axis_name="s")
    return pl.kernel(
        _sc_gather,
        out_type=jax.ShapeDtypeStruct((BATCH, HIDDEN), jnp.float32),
        mesh=mesh,
        scratch_types=[
            pltpu.VMEM((B_PER_W,), jnp.int32),
            pltpu.VMEM((B_PER_W, HIDDEN), jnp.float32),
            pltpu.SemaphoreType.DMA,
            pltpu.SemaphoreType.DMA,
        ],
        compiler_params=pltpu.CompilerParams(needs_layout_passes=False, use_tc_tiling_on_sc=True),
    )(tidx, table)


def _mlp_body(emb_ref, w1_ref, b1_ref, w2_ref, b2_ref, gamma_ref, beta_ref,
              out_ref):
    x = emb_ref[...]
    h = jnp.dot(x, w1_ref[...], preferred_element_type=jnp.float32)
    h = h + b1_ref[...]
    h = 0.5 * h * (1.0 + lax.erf(h * 0.7071067811865476))
    y = jnp.dot(h, w2_ref[...], preferred_element_type=jnp.float32)
    y = y + b2_ref[...]
    mean = jnp.mean(y, axis=-1, keepdims=True)
    c = y - mean
    var = jnp.mean(c * c, axis=-1, keepdims=True)
    out_ref[...] = c * lax.rsqrt(var + 1e-5) * gamma_ref[...] + beta_ref[...]


@jax.jit
def _mlp(emb, W1, b1, W2, b2, gamma, beta):
    blk = 2048
    grid = (BATCH // blk,)
    rep2 = lambda i: (0, 0)
    return pl.pallas_call(
        _mlp_body,
        grid=grid,
        in_specs=[
            pl.BlockSpec((blk, HIDDEN), lambda i: (i, 0)),
            pl.BlockSpec((HIDDEN, 2 * HIDDEN), rep2),
            pl.BlockSpec((1, 2 * HIDDEN), rep2),
            pl.BlockSpec((2 * HIDDEN, HIDDEN), rep2),
            pl.BlockSpec((1, HIDDEN), rep2),
            pl.BlockSpec((1, HIDDEN), rep2),
            pl.BlockSpec((1, HIDDEN), rep2),
        ],
        out_specs=pl.BlockSpec((blk, HIDDEN), lambda i: (i, 0)),
        out_shape=jax.ShapeDtypeStruct((BATCH, HIDDEN), jnp.float32),
    )(emb, W1, b1.reshape(1, -1), W2, b2.reshape(1, -1),
      gamma.reshape(1, -1), beta.reshape(1, -1))


def kernel(class_labels, table, W1, b1, W2, b2, gamma, beta):
    emb = _gather_rows(class_labels, table)
    return _mlp(emb, W1, b1, W2, b2, gamma, beta)


# speedup vs baseline: 1.0042x; 1.0042x over previous
import jax
import jax.numpy as jnp
from jax import lax
from jax.experimental import pallas as pl
from jax.experimental.pallas import tpu as pltpu
from jax.experimental.pallas import tpu_sc as plsc

BATCH = 16384
HIDDEN = 64
NC = 2
NS = 16
NW = NC * NS
B_PER_W = BATCH // NW


def _sc_gather(tidx_hbm, tbl_hbm, out_hbm, idx_v, rows_v, gsem, wsem):
    wid = lax.axis_index("s") * NC + lax.axis_index("c")
    base = wid * B_PER_W
    pltpu.sync_copy(tidx_hbm.at[wid], idx_v)

    def fire_group(g, _):
        vec = idx_v[pl.ds(g * 16, 16)]
        for lane in range(16):
            pltpu.make_async_copy(
                tbl_hbm.at[pl.ds(vec[lane], 1), :],
                rows_v.at[pl.ds(g * 16 + lane, 1), :],
                gsem,
            ).start()
        return _

    lax.fori_loop(0, B_PER_W // 16, fire_group, None)
    pltpu.make_async_copy(
        tbl_hbm.at[pl.ds(0, B_PER_W), :], rows_v, gsem).wait()
    pltpu.sync_copy(rows_v, out_hbm.at[pl.ds(base, B_PER_W)])


@jax.jit
def _gather_rows(class_labels, table):
    tidx = class_labels.astype(jnp.int32).reshape(NW, B_PER_W)
    mesh = plsc.VectorSubcoreMesh(core_axis_name="c", subcore_axis_name="s")
    return pl.kernel(
        _sc_gather,
        out_type=jax.ShapeDtypeStruct((BATCH, HIDDEN), jnp.float32),
        mesh=mesh,
        scratch_types=[
            pltpu.VMEM((B_PER_W,), jnp.int32),
            pltpu.VMEM((B_PER_W, HIDDEN), jnp.float32),
            pltpu.SemaphoreType.DMA,
            pltpu.SemaphoreType.DMA,
        ],
    )(tidx, table)


def _mlp_body(emb_ref, w1_ref, b1_ref, w2_ref, b2_ref, gamma_ref, beta_ref,
              out_ref):
    x = emb_ref[...]
    h = jnp.dot(x, w1_ref[...], preferred_element_type=jnp.float32)
    h = h + b1_ref[...]
    h = 0.5 * h * (1.0 + lax.erf(h * 0.7071067811865476))
    y = jnp.dot(h, w2_ref[...], preferred_element_type=jnp.float32)
    y = y + b2_ref[...]
    mean = jnp.mean(y, axis=-1, keepdims=True)
    c = y - mean
    var = jnp.mean(c * c, axis=-1, keepdims=True)
    out_ref[...] = c * lax.rsqrt(var + 1e-5) * gamma_ref[...] + beta_ref[...]


@jax.jit
def _mlp(emb, W1, b1, W2, b2, gamma, beta):
    blk = 2048
    grid = (BATCH // blk,)
    rep2 = lambda i: (0, 0)
    return pl.pallas_call(
        _mlp_body,
        grid=grid,
        in_specs=[
            pl.BlockSpec((blk, HIDDEN), lambda i: (i, 0)),
            pl.BlockSpec((HIDDEN, 2 * HIDDEN), rep2),
            pl.BlockSpec((1, 2 * HIDDEN), rep2),
            pl.BlockSpec((2 * HIDDEN, HIDDEN), rep2),
            pl.BlockSpec((1, HIDDEN), rep2),
            pl.BlockSpec((1, HIDDEN), rep2),
            pl.BlockSpec((1, HIDDEN), rep2),
        ],
        out_specs=pl.BlockSpec((blk, HIDDEN), lambda i: (i, 0)),
        out_shape=jax.ShapeDtypeStruct((BATCH, HIDDEN), jnp.float32),
    )(emb, W1, b1.reshape(1, -1), W2, b2.reshape(1, -1),
      gamma.reshape(1, -1), beta.reshape(1, -1))


def kernel(class_labels, table, W1, b1, W2, b2, gamma, beta):
    emb = _gather_rows(class_labels, table)
    return _mlp(emb, W1, b1, W2, b2, gamma, beta)


# speedup vs baseline: 1.2871x; 1.2817x over previous
import jax
import jax.numpy as jnp
from jax import lax
from jax.experimental import pallas as pl
from jax.experimental.pallas import tpu as pltpu
from jax.experimental.pallas import tpu_sc as plsc

BATCH = 16384
HIDDEN = 64
NC = 2
NS = 16
NW = NC * NS
B_PER_W = BATCH // NW
LISTW = B_PER_W + 16


def _sc_gather(bids_hbm, cnts_hbm, lanes_hbm, dests_hbm, nb_hbm, tblT_hbm,
               out_hbm, bids_v, cnts_v, lanes_v, dests_v, nb_v,
               buf0, buf1, rows_v, bsem0, bsem1, osem):
    wid = lax.axis_index("s") * NC + lax.axis_index("c")
    pltpu.sync_copy(bids_hbm.at[wid], bids_v)
    pltpu.sync_copy(cnts_hbm.at[wid], cnts_v)
    pltpu.sync_copy(lanes_hbm.at[wid], lanes_v)
    pltpu.sync_copy(dests_hbm.at[wid], dests_v)
    pltpu.sync_copy(nb_hbm.at[wid], nb_v)

    n_blocks = nb_v[...][0]
    ngrp = (n_blocks + 15) // 16
    iota16 = lax.iota(jnp.int32, 16)
    bufs = (buf0, buf1)
    bsems = (bsem0, bsem1)

    def block_dma(bid, lane):
        off = pl.multiple_of(bid * 128, 128)
        return pltpu.make_async_copy(
            tblT_hbm.at[:, pl.ds(off, 128)], bufs[lane % 2], bsems[lane % 2])

    def process_block(buf, cnt, cur):
        def lab(q, c):
            spos = cur + q
            lo = lanes_v[pl.ds(spos, 16)][0]
            dest = dests_v[pl.ds(spos, 16)][0]
            cvec = jnp.full((16,), lo, jnp.int32)
            svec = jnp.full((16,), spos, jnp.int32)
            for m in range(4):
                ridx = iota16 + (16 * m)
                col = plsc.load_gather(buf, [ridx, cvec])
                plsc.store_scatter(rows_v, [svec, ridx], col)
            pltpu.make_async_copy(
                rows_v.at[pl.ds(spos, 1), :],
                out_hbm.at[pl.ds(dest, 1), :],
                osem,
            ).start()
            return c

        lax.fori_loop(0, cnt, lab, jnp.int32(0))
        return cur + cnt

    def group(G, cursor):
        b16 = G * 16
        bvec = bids_v[pl.ds(b16, 16)]
        cvec = cnts_v[pl.ds(b16, 16)]
        handles = [None] * 16
        for lane in (0, 1):
            handles[lane] = block_dma(bvec[lane], lane)
            handles[lane].start()
        cur = cursor
        for lane in range(16):
            handles[lane].wait()
            cur = process_block(bufs[lane % 2], cvec[lane], cur)
            nxt = lane + 2
            if nxt < 16:
                handles[nxt] = block_dma(bvec[nxt], nxt)
                handles[nxt].start()
        return cur

    lax.fori_loop(0, ngrp, group, jnp.int32(0))
    pltpu.make_async_copy(
        out_hbm.at[pl.ds(0, B_PER_W), :], rows_v, osem).wait()


@jax.jit
def _gather_rows(class_labels, table):
    labels = class_labels.astype(jnp.int32)
    order = jnp.argsort(labels).astype(jnp.int32)
    sl = jnp.take(labels, order)
    blk = sl >> 7
    pos = jnp.arange(BATCH, dtype=jnp.int32)
    seg = pos // B_PER_W
    new = ((pos % B_PER_W) == 0) | (blk != jnp.roll(blk, 1))
    k = jnp.cumsum(new.astype(jnp.int32)) - 1
    ks = k - jnp.take(k, seg * B_PER_W)
    bids = jnp.zeros((NW, B_PER_W), jnp.int32).at[seg, ks].set(blk)
    cnts = jnp.zeros((NW, B_PER_W), jnp.int32).at[seg, ks].add(1)
    nb = ks.reshape(NW, B_PER_W)[:, -1] + 1
    pad2 = lambda a: jnp.pad(a, ((0, 0), (0, 16)))
    bids = pad2(bids)
    cnts = pad2(cnts)
    lanes = pad2((sl & 127).reshape(NW, B_PER_W))
    dests = pad2(order.reshape(NW, B_PER_W))
    nb16 = jnp.tile(nb[:, None], (1, 16))

    mesh = plsc.VectorSubcoreMesh(core_axis_name="c", subcore_axis_name="s")
    return pl.kernel(
        _sc_gather,
        out_type=jax.ShapeDtypeStruct((BATCH, HIDDEN), jnp.float32),
        mesh=mesh,
        scratch_types=[
            pltpu.VMEM((LISTW,), jnp.int32),
            pltpu.VMEM((LISTW,), jnp.int32),
            pltpu.VMEM((LISTW,), jnp.int32),
            pltpu.VMEM((LISTW,), jnp.int32),
            pltpu.VMEM((16,), jnp.int32),
            pltpu.VMEM((HIDDEN, 128), jnp.float32),
            pltpu.VMEM((HIDDEN, 128), jnp.float32),
            pltpu.VMEM((B_PER_W, HIDDEN), jnp.float32),
            pltpu.SemaphoreType.DMA,
            pltpu.SemaphoreType.DMA,
            pltpu.SemaphoreType.DMA,
        ],
        compiler_params=pltpu.CompilerParams(needs_layout_passes=False),
    )(bids, cnts, lanes, dests, nb16, table.T)


def _mlp_body(emb_ref, w1_ref, b1_ref, w2_ref, b2_ref, gamma_ref, beta_ref,
              out_ref):
    x = emb_ref[...]
    h = jnp.dot(x, w1_ref[...], preferred_element_type=jnp.float32)
    h = h + b1_ref[...]
    h = 0.5 * h * (1.0 + lax.erf(h * 0.7071067811865476))
    y = jnp.dot(h, w2_ref[...], preferred_element_type=jnp.float32)
    y = y + b2_ref[...]
    mean = jnp.mean(y, axis=-1, keepdims=True)
    c = y - mean
    var = jnp.mean(c * c, axis=-1, keepdims=True)
    out_ref[...] = c * lax.rsqrt(var + 1e-5) * gamma_ref[...] + beta_ref[...]


@jax.jit
def _mlp(emb, W1, b1, W2, b2, gamma, beta):
    blk = 2048
    grid = (BATCH // blk,)
    rep2 = lambda i: (0, 0)
    return pl.pallas_call(
        _mlp_body,
        grid=grid,
        in_specs=[
            pl.BlockSpec((blk, HIDDEN), lambda i: (i, 0)),
            pl.BlockSpec((HIDDEN, 2 * HIDDEN), rep2),
            pl.BlockSpec((1, 2 * HIDDEN), rep2),
            pl.BlockSpec((2 * HIDDEN, HIDDEN), rep2),
            pl.BlockSpec((1, HIDDEN), rep2),
            pl.BlockSpec((1, HIDDEN), rep2),
            pl.BlockSpec((1, HIDDEN), rep2),
        ],
        out_specs=pl.BlockSpec((blk, HIDDEN), lambda i: (i, 0)),
        out_shape=jax.ShapeDtypeStruct((BATCH, HIDDEN), jnp.float32),
    )(emb, W1, b1.reshape(1, -1), W2, b2.reshape(1, -1),
      gamma.reshape(1, -1), beta.reshape(1, -1))


def kernel(class_labels, table, W1, b1, W2, b2, gamma, beta):
    emb = _gather_rows(class_labels, table)
    return _mlp(emb, W1, b1, W2, b2, gamma, beta)


# speedup vs baseline: 1.9543x; 1.5183x over previous
import jax
import jax.numpy as jnp
from jax import lax
from jax.experimental import pallas as pl
from jax.experimental.pallas import tpu as pltpu
from jax.experimental.pallas import tpu_sc as plsc

BATCH = 16384
HIDDEN = 64
NC = 2
NS = 16
NW = NC * NS
B_PER_W = BATCH // NW
LISTW = B_PER_W + 16


def _sc_gather(bids_hbm, cnts_hbm, lanes_hbm, dests_hbm, nb_hbm, tblT_hbm,
               out_hbm, bids_v, cnts_v, lanes_v, dests_v, nb_v,
               buf0, buf1, buf2, buf3, rows_v, bsem0, bsem1, bsem2, bsem3,
               osem):
    wid = lax.axis_index("s") * NC + lax.axis_index("c")
    pltpu.sync_copy(bids_hbm.at[wid], bids_v)
    pltpu.sync_copy(cnts_hbm.at[wid], cnts_v)
    pltpu.sync_copy(lanes_hbm.at[wid], lanes_v)
    pltpu.sync_copy(dests_hbm.at[wid], dests_v)
    pltpu.sync_copy(nb_hbm.at[wid], nb_v)

    n_blocks = nb_v[...][0]
    ngrp = (n_blocks + 15) // 16
    iota16 = lax.iota(jnp.int32, 16)
    bufs = (buf0, buf1, buf2, buf3)
    bsems = (bsem0, bsem1, bsem2, bsem3)

    def block_dma(bid, lane):
        off = pl.multiple_of(bid * 128, 128)
        return pltpu.make_async_copy(
            tblT_hbm.at[:, pl.ds(off, 128)], bufs[lane % 4], bsems[lane % 4])

    def process_block(buf, cnt, cur):
        def lab(q, c):
            spos = cur + q
            lo = lanes_v[pl.ds(spos, 16)][0]
            dest = dests_v[pl.ds(spos, 16)][0]
            cvec = jnp.full((16,), lo, jnp.int32)
            svec = jnp.full((16,), spos, jnp.int32)
            for m in range(4):
                ridx = iota16 + (16 * m)
                col = plsc.load_gather(buf, [ridx, cvec])
                plsc.store_scatter(rows_v, [svec, ridx], col)
            pltpu.make_async_copy(
                rows_v.at[pl.ds(spos, 1), :],
                out_hbm.at[pl.ds(dest, 1), :],
                osem,
            ).start()
            return c

        lax.fori_loop(0, cnt, lab, jnp.int32(0))
        return cur + cnt

    def group(G, cursor):
        b16 = G * 16
        bvec = bids_v[pl.ds(b16, 16)]
        cvec = cnts_v[pl.ds(b16, 16)]
        handles = [None] * 16
        for lane in (0, 1, 2, 3):
            handles[lane] = block_dma(bvec[lane], lane)
            handles[lane].start()
        cur = cursor
        for lane in range(16):
            handles[lane].wait()
            cur = process_block(bufs[lane % 4], cvec[lane], cur)
            nxt = lane + 4
            if nxt < 16:
                handles[nxt] = block_dma(bvec[nxt], nxt)
                handles[nxt].start()
        return cur

    lax.fori_loop(0, ngrp, group, jnp.int32(0))
    pltpu.make_async_copy(
        out_hbm.at[pl.ds(0, B_PER_W), :], rows_v, osem).wait()


@jax.jit
def _gather_rows(class_labels, table):
    labels = class_labels.astype(jnp.int32)
    order = jnp.argsort(labels).astype(jnp.int32)
    sl = jnp.take(labels, order)
    blk = sl >> 7
    pos = jnp.arange(BATCH, dtype=jnp.int32)
    seg = pos // B_PER_W
    new = ((pos % B_PER_W) == 0) | (blk != jnp.roll(blk, 1))
    key = seg * 1024 + (1 - new.astype(jnp.int32)) * 512 + (pos % B_PER_W)
    P = jnp.argsort(key).astype(jnp.int32).reshape(NW, B_PER_W)
    nb = jnp.sum(new.reshape(NW, B_PER_W), axis=1).astype(jnp.int32)
    j2 = jnp.arange(B_PER_W, dtype=jnp.int32)[None, :]
    valid = j2 < nb[:, None]
    Pn = jnp.roll(P, -1, axis=1)
    segend = (jnp.arange(NW, dtype=jnp.int32)[:, None] + 1) * B_PER_W
    cnts = jnp.where(j2 + 1 < nb[:, None], Pn - P, segend - P)
    cnts = jnp.where(valid, cnts, 0)
    bids = jnp.where(valid, jnp.take(blk, jnp.minimum(P, BATCH - 1)), 0)
    pad2 = lambda a: jnp.pad(a, ((0, 0), (0, 16)))
    bids = pad2(bids)
    cnts = pad2(cnts)
    lanes = pad2((sl & 127).reshape(NW, B_PER_W))
    dests = pad2(order.reshape(NW, B_PER_W))
    nb16 = jnp.tile(nb[:, None], (1, 16))

    mesh = plsc.VectorSubcoreMesh(core_axis_name="c", subcore_axis_name="s")
    return pl.kernel(
        _sc_gather,
        out_type=jax.ShapeDtypeStruct((BATCH, HIDDEN), jnp.float32),
        mesh=mesh,
        scratch_types=[
            pltpu.VMEM((LISTW,), jnp.int32),
            pltpu.VMEM((LISTW,), jnp.int32),
            pltpu.VMEM((LISTW,), jnp.int32),
            pltpu.VMEM((LISTW,), jnp.int32),
            pltpu.VMEM((16,), jnp.int32),
            pltpu.VMEM((HIDDEN, 128), jnp.float32),
            pltpu.VMEM((HIDDEN, 128), jnp.float32),
            pltpu.VMEM((HIDDEN, 128), jnp.float32),
            pltpu.VMEM((HIDDEN, 128), jnp.float32),
            pltpu.VMEM((B_PER_W, HIDDEN), jnp.float32),
            pltpu.SemaphoreType.DMA,
            pltpu.SemaphoreType.DMA,
            pltpu.SemaphoreType.DMA,
            pltpu.SemaphoreType.DMA,
            pltpu.SemaphoreType.DMA,
        ],
        compiler_params=pltpu.CompilerParams(needs_layout_passes=False),
    )(bids, cnts, lanes, dests, nb16, table.T)


def _mlp_body(emb_ref, w1_ref, b1_ref, w2_ref, b2_ref, gamma_ref, beta_ref,
              out_ref):
    x = emb_ref[...]
    h = jnp.dot(x, w1_ref[...], preferred_element_type=jnp.float32)
    h = h + b1_ref[...]
    h = 0.5 * h * (1.0 + lax.erf(h * 0.7071067811865476))
    y = jnp.dot(h, w2_ref[...], preferred_element_type=jnp.float32)
    y = y + b2_ref[...]
    mean = jnp.mean(y, axis=-1, keepdims=True)
    c = y - mean
    var = jnp.mean(c * c, axis=-1, keepdims=True)
    out_ref[...] = c * lax.rsqrt(var + 1e-5) * gamma_ref[...] + beta_ref[...]


@jax.jit
def _mlp(emb, W1, b1, W2, b2, gamma, beta):
    blk = 2048
    grid = (BATCH // blk,)
    rep2 = lambda i: (0, 0)
    return pl.pallas_call(
        _mlp_body,
        grid=grid,
        in_specs=[
            pl.BlockSpec((blk, HIDDEN), lambda i: (i, 0)),
            pl.BlockSpec((HIDDEN, 2 * HIDDEN), rep2),
            pl.BlockSpec((1, 2 * HIDDEN), rep2),
            pl.BlockSpec((2 * HIDDEN, HIDDEN), rep2),
            pl.BlockSpec((1, HIDDEN), rep2),
            pl.BlockSpec((1, HIDDEN), rep2),
            pl.BlockSpec((1, HIDDEN), rep2),
        ],
        out_specs=pl.BlockSpec((blk, HIDDEN), lambda i: (i, 0)),
        out_shape=jax.ShapeDtypeStruct((BATCH, HIDDEN), jnp.float32),
    )(emb, W1, b1.reshape(1, -1), W2, b2.reshape(1, -1),
      gamma.reshape(1, -1), beta.reshape(1, -1))


def kernel(class_labels, table, W1, b1, W2, b2, gamma, beta):
    emb = _gather_rows(class_labels, table)
    return _mlp(emb, W1, b1, W2, b2, gamma, beta)


# speedup vs baseline: 2.0832x; 1.0660x over previous
import jax
import jax.numpy as jnp
from jax import lax
from jax.experimental import pallas as pl
from jax.experimental.pallas import tpu as pltpu
from jax.experimental.pallas import tpu_sc as plsc

BATCH = 16384
HIDDEN = 64
NC = 2
NS = 16
NW = NC * NS
B_PER_W = BATCH // NW
LISTW = B_PER_W + 16


def _sc_gather(bids_hbm, cnts_hbm, lanes_hbm, dests_hbm, nb_hbm, tblT_hbm,
               out_hbm, bids_v, cnts_v, lanes_v, dests_v, nb_v,
               buf0, buf1, buf2, buf3, rows_v, bsem0, bsem1, bsem2, bsem3,
               osem):
    wid = lax.axis_index("s") * NC + lax.axis_index("c")
    pltpu.sync_copy(bids_hbm.at[wid], bids_v)
    pltpu.sync_copy(cnts_hbm.at[wid], cnts_v)
    pltpu.sync_copy(lanes_hbm.at[wid], lanes_v)
    pltpu.sync_copy(dests_hbm.at[wid], dests_v)
    pltpu.sync_copy(nb_hbm.at[wid], nb_v)

    n_blocks = nb_v[...][0]
    ngrp = (n_blocks + 15) // 16
    iota16 = lax.iota(jnp.int32, 16)
    bufs = (buf0, buf1, buf2, buf3)
    bsems = (bsem0, bsem1, bsem2, bsem3)

    def block_dma(bid, lane):
        off = pl.multiple_of(bid * 128, 128)
        return pltpu.make_async_copy(
            tblT_hbm.at[:, pl.ds(off, 128)], bufs[lane % 4], bsems[lane % 4])

    def process_block(buf, cnt, cur):
        def lab(q, c):
            spos = cur + q
            lo = lanes_v[pl.ds(spos, 16)][0]
            dest = dests_v[pl.ds(spos, 16)][0]
            cvec = jnp.full((16,), lo, jnp.int32)
            svec = jnp.full((16,), spos, jnp.int32)
            for m in range(4):
                ridx = iota16 + (16 * m)
                col = plsc.load_gather(buf, [ridx, cvec])
                plsc.store_scatter(rows_v, [svec, ridx], col)
            pltpu.make_async_copy(
                rows_v.at[pl.ds(spos, 1), :],
                out_hbm.at[pl.ds(dest, 1), :],
                osem,
            ).start()
            return c

        lax.fori_loop(0, cnt, lab, jnp.int32(0))
        return cur + cnt

    def group(G, cursor):
        b16 = G * 16
        bvec = bids_v[pl.ds(b16, 16)]
        cvec = cnts_v[pl.ds(b16, 16)]
        handles = [None] * 16
        for lane in (0, 1, 2, 3):
            handles[lane] = block_dma(bvec[lane], lane)
            handles[lane].start()
        cur = cursor
        for lane in range(16):
            handles[lane].wait()
            cur = process_block(bufs[lane % 4], cvec[lane], cur)
            nxt = lane + 4
            if nxt < 16:
                handles[nxt] = block_dma(bvec[nxt], nxt)
                handles[nxt].start()
        return cur

    lax.fori_loop(0, ngrp, group, jnp.int32(0))
    pltpu.make_async_copy(
        out_hbm.at[pl.ds(0, B_PER_W), :], rows_v, osem).wait()


@jax.jit
def _gather_rows(class_labels, table):
    labels = class_labels.astype(jnp.int32)
    pos = jnp.arange(BATCH, dtype=jnp.int32)
    sl, order = lax.sort((labels, pos), num_keys=1)
    blk = sl >> 7
    seg = pos // B_PER_W
    new = ((pos % B_PER_W) == 0) | (blk != jnp.roll(blk, 1))
    key = seg * 1024 + (1 - new.astype(jnp.int32)) * 512 + (pos % B_PER_W)
    _, Pf, blkP = lax.sort((key, pos, blk), num_keys=1)
    P = Pf.reshape(NW, B_PER_W)
    nb = jnp.sum(new.reshape(NW, B_PER_W), axis=1).astype(jnp.int32)
    j2 = jnp.arange(B_PER_W, dtype=jnp.int32)[None, :]
    valid = j2 < nb[:, None]
    Pn = jnp.roll(P, -1, axis=1)
    segend = (jnp.arange(NW, dtype=jnp.int32)[:, None] + 1) * B_PER_W
    cnts = jnp.where(j2 + 1 < nb[:, None], Pn - P, segend - P)
    cnts = jnp.where(valid, cnts, 0)
    bids = jnp.where(valid, blkP.reshape(NW, B_PER_W), 0)
    pad2 = lambda a: jnp.pad(a, ((0, 0), (0, 16)))
    bids = pad2(bids)
    cnts = pad2(cnts)
    lanes = pad2((sl & 127).reshape(NW, B_PER_W))
    dests = pad2(order.reshape(NW, B_PER_W))
    nb16 = jnp.tile(nb[:, None], (1, 16))

    mesh = plsc.VectorSubcoreMesh(core_axis_name="c", subcore_axis_name="s")
    return pl.kernel(
        _sc_gather,
        out_type=jax.ShapeDtypeStruct((BATCH, HIDDEN), jnp.float32),
        mesh=mesh,
        scratch_types=[
            pltpu.VMEM((LISTW,), jnp.int32),
            pltpu.VMEM((LISTW,), jnp.int32),
            pltpu.VMEM((LISTW,), jnp.int32),
            pltpu.VMEM((LISTW,), jnp.int32),
            pltpu.VMEM((16,), jnp.int32),
            pltpu.VMEM((HIDDEN, 128), jnp.float32),
            pltpu.VMEM((HIDDEN, 128), jnp.float32),
            pltpu.VMEM((HIDDEN, 128), jnp.float32),
            pltpu.VMEM((HIDDEN, 128), jnp.float32),
            pltpu.VMEM((B_PER_W, HIDDEN), jnp.float32),
            pltpu.SemaphoreType.DMA,
            pltpu.SemaphoreType.DMA,
            pltpu.SemaphoreType.DMA,
            pltpu.SemaphoreType.DMA,
            pltpu.SemaphoreType.DMA,
        ],
        compiler_params=pltpu.CompilerParams(needs_layout_passes=False),
    )(bids, cnts, lanes, dests, nb16, table.T)


def _mlp_body(emb_ref, w1_ref, b1_ref, w2_ref, b2_ref, gamma_ref, beta_ref,
              out_ref):
    x = emb_ref[...]
    h = jnp.dot(x, w1_ref[...], preferred_element_type=jnp.float32)
    h = h + b1_ref[...]
    h = 0.5 * h * (1.0 + lax.erf(h * 0.7071067811865476))
    y = jnp.dot(h, w2_ref[...], preferred_element_type=jnp.float32)
    y = y + b2_ref[...]
    mean = jnp.mean(y, axis=-1, keepdims=True)
    c = y - mean
    var = jnp.mean(c * c, axis=-1, keepdims=True)
    out_ref[...] = c * lax.rsqrt(var + 1e-5) * gamma_ref[...] + beta_ref[...]


@jax.jit
def _mlp(emb, W1, b1, W2, b2, gamma, beta):
    blk = 2048
    grid = (BATCH // blk,)
    rep2 = lambda i: (0, 0)
    return pl.pallas_call(
        _mlp_body,
        grid=grid,
        in_specs=[
            pl.BlockSpec((blk, HIDDEN), lambda i: (i, 0)),
            pl.BlockSpec((HIDDEN, 2 * HIDDEN), rep2),
            pl.BlockSpec((1, 2 * HIDDEN), rep2),
            pl.BlockSpec((2 * HIDDEN, HIDDEN), rep2),
            pl.BlockSpec((1, HIDDEN), rep2),
            pl.BlockSpec((1, HIDDEN), rep2),
            pl.BlockSpec((1, HIDDEN), rep2),
        ],
        out_specs=pl.BlockSpec((blk, HIDDEN), lambda i: (i, 0)),
        out_shape=jax.ShapeDtypeStruct((BATCH, HIDDEN), jnp.float32),
    )(emb, W1, b1.reshape(1, -1), W2, b2.reshape(1, -1),
      gamma.reshape(1, -1), beta.reshape(1, -1))


def kernel(class_labels, table, W1, b1, W2, b2, gamma, beta):
    emb = _gather_rows(class_labels, table)
    return _mlp(emb, W1, b1, W2, b2, gamma, beta)


# speedup vs baseline: 2.1792x; 1.0461x over previous
import jax
import jax.numpy as jnp
from jax import lax
from jax.experimental import pallas as pl
from jax.experimental.pallas import tpu as pltpu
from jax.experimental.pallas import tpu_sc as plsc

BATCH = 16384
HIDDEN = 64
NC = 2
NS = 16
NW = NC * NS
B_PER_W = BATCH // NW
LISTW = B_PER_W + 16


def _sc_gather(bids_hbm, cnts_hbm, lanes_hbm, dests_hbm, nb_hbm, tblT_hbm,
               out_hbm, bids_v, cnts_v, lanes_v, dests_v, nb_v,
               buf0, buf1, buf2, buf3, buf4, buf5, rows_v,
               bsem0, bsem1, bsem2, bsem3, bsem4, bsem5, osem):
    wid = lax.axis_index("s") * NC + lax.axis_index("c")
    pltpu.sync_copy(bids_hbm.at[wid], bids_v)
    pltpu.sync_copy(cnts_hbm.at[wid], cnts_v)
    pltpu.sync_copy(lanes_hbm.at[wid], lanes_v)
    pltpu.sync_copy(dests_hbm.at[wid], dests_v)
    pltpu.sync_copy(nb_hbm.at[wid], nb_v)

    n_blocks = nb_v[...][0]
    ngrp = (n_blocks + 15) // 16
    iota16 = lax.iota(jnp.int32, 16)
    bufs = (buf0, buf1, buf2, buf3, buf4, buf5)
    bsems = (bsem0, bsem1, bsem2, bsem3, bsem4, bsem5)

    def block_dma(bid, lane):
        off = pl.multiple_of(bid * 128, 128)
        return pltpu.make_async_copy(
            tblT_hbm.at[:, pl.ds(off, 128)], bufs[lane % 6], bsems[lane % 6])

    def process_block(buf, cnt, cur):
        def lab(q, c):
            spos = cur + q
            lo = lanes_v[pl.ds(spos, 16)][0]
            dest = dests_v[pl.ds(spos, 16)][0]
            cvec = jnp.full((16,), lo, jnp.int32)
            svec = jnp.full((16,), spos, jnp.int32)
            for m in range(4):
                ridx = iota16 + (16 * m)
                col = plsc.load_gather(buf, [ridx, cvec])
                plsc.store_scatter(rows_v, [svec, ridx], col)
            pltpu.make_async_copy(
                rows_v.at[pl.ds(spos, 1), :],
                out_hbm.at[pl.ds(dest, 1), :],
                osem,
            ).start()
            return c

        lax.fori_loop(0, cnt, lab, jnp.int32(0))
        return cur + cnt

    def group(G, cursor):
        b16 = G * 16
        bvec = bids_v[pl.ds(b16, 16)]
        cvec = cnts_v[pl.ds(b16, 16)]
        handles = [None] * 16
        for lane in (0, 1, 2, 3, 4, 5):
            handles[lane] = block_dma(bvec[lane], lane)
            handles[lane].start()
        cur = cursor
        for lane in range(16):
            handles[lane].wait()
            cur = process_block(bufs[lane % 6], cvec[lane], cur)
            nxt = lane + 6
            if nxt < 16:
                handles[nxt] = block_dma(bvec[nxt], nxt)
                handles[nxt].start()
        return cur

    lax.fori_loop(0, ngrp, group, jnp.int32(0))
    pltpu.make_async_copy(
        out_hbm.at[pl.ds(0, B_PER_W), :], rows_v, osem).wait()


@jax.jit
def _gather_rows(class_labels, table):
    labels = class_labels.astype(jnp.int32)
    pos = jnp.arange(BATCH, dtype=jnp.int32)
    sl, order = lax.sort((labels, pos), num_keys=1)
    blk = sl >> 7
    seg = pos // B_PER_W
    new = ((pos % B_PER_W) == 0) | (blk != jnp.roll(blk, 1))
    key = seg * 1024 + (1 - new.astype(jnp.int32)) * 512 + (pos % B_PER_W)
    _, Pf, blkP = lax.sort((key, pos, blk), num_keys=1)
    P = Pf.reshape(NW, B_PER_W)
    nb = jnp.sum(new.reshape(NW, B_PER_W), axis=1).astype(jnp.int32)
    j2 = jnp.arange(B_PER_W, dtype=jnp.int32)[None, :]
    valid = j2 < nb[:, None]
    Pn = jnp.roll(P, -1, axis=1)
    segend = (jnp.arange(NW, dtype=jnp.int32)[:, None] + 1) * B_PER_W
    cnts = jnp.where(j2 + 1 < nb[:, None], Pn - P, segend - P)
    cnts = jnp.where(valid, cnts, 0)
    bids = jnp.where(valid, blkP.reshape(NW, B_PER_W), 0)
    pad2 = lambda a: jnp.pad(a, ((0, 0), (0, 16)))
    bids = pad2(bids)
    cnts = pad2(cnts)
    lanes = pad2((sl & 127).reshape(NW, B_PER_W))
    dests = pad2(order.reshape(NW, B_PER_W))
    nb16 = jnp.tile(nb[:, None], (1, 16))

    mesh = plsc.VectorSubcoreMesh(core_axis_name="c", subcore_axis_name="s")
    return pl.kernel(
        _sc_gather,
        out_type=jax.ShapeDtypeStruct((BATCH, HIDDEN), jnp.float32),
        mesh=mesh,
        scratch_types=[
            pltpu.VMEM((LISTW,), jnp.int32),
            pltpu.VMEM((LISTW,), jnp.int32),
            pltpu.VMEM((LISTW,), jnp.int32),
            pltpu.VMEM((LISTW,), jnp.int32),
            pltpu.VMEM((16,), jnp.int32),
            pltpu.VMEM((HIDDEN, 128), jnp.float32),
            pltpu.VMEM((HIDDEN, 128), jnp.float32),
            pltpu.VMEM((HIDDEN, 128), jnp.float32),
            pltpu.VMEM((HIDDEN, 128), jnp.float32),
            pltpu.VMEM((HIDDEN, 128), jnp.float32),
            pltpu.VMEM((HIDDEN, 128), jnp.float32),
            pltpu.VMEM((B_PER_W, HIDDEN), jnp.float32),
            pltpu.SemaphoreType.DMA,
            pltpu.SemaphoreType.DMA,
            pltpu.SemaphoreType.DMA,
            pltpu.SemaphoreType.DMA,
            pltpu.SemaphoreType.DMA,
            pltpu.SemaphoreType.DMA,
            pltpu.SemaphoreType.DMA,
        ],
        compiler_params=pltpu.CompilerParams(needs_layout_passes=False),
    )(bids, cnts, lanes, dests, nb16, table.T)


def _mlp_body(emb_ref, w1_ref, b1_ref, w2_ref, b2_ref, gamma_ref, beta_ref,
              out_ref):
    x = emb_ref[...]
    h = jnp.dot(x, w1_ref[...], preferred_element_type=jnp.float32)
    h = h + b1_ref[...]
    h = 0.5 * h * (1.0 + lax.erf(h * 0.7071067811865476))
    y = jnp.dot(h, w2_ref[...], preferred_element_type=jnp.float32)
    y = y + b2_ref[...]
    mean = jnp.mean(y, axis=-1, keepdims=True)
    c = y - mean
    var = jnp.mean(c * c, axis=-1, keepdims=True)
    out_ref[...] = c * lax.rsqrt(var + 1e-5) * gamma_ref[...] + beta_ref[...]


@jax.jit
def _mlp(emb, W1, b1, W2, b2, gamma, beta):
    blk = 2048
    grid = (BATCH // blk,)
    rep2 = lambda i: (0, 0)
    return pl.pallas_call(
        _mlp_body,
        grid=grid,
        in_specs=[
            pl.BlockSpec((blk, HIDDEN), lambda i: (i, 0)),
            pl.BlockSpec((HIDDEN, 2 * HIDDEN), rep2),
            pl.BlockSpec((1, 2 * HIDDEN), rep2),
            pl.BlockSpec((2 * HIDDEN, HIDDEN), rep2),
            pl.BlockSpec((1, HIDDEN), rep2),
            pl.BlockSpec((1, HIDDEN), rep2),
            pl.BlockSpec((1, HIDDEN), rep2),
        ],
        out_specs=pl.BlockSpec((blk, HIDDEN), lambda i: (i, 0)),
        out_shape=jax.ShapeDtypeStruct((BATCH, HIDDEN), jnp.float32),
    )(emb, W1, b1.reshape(1, -1), W2, b2.reshape(1, -1),
      gamma.reshape(1, -1), beta.reshape(1, -1))


def kernel(class_labels, table, W1, b1, W2, b2, gamma, beta):
    emb = _gather_rows(class_labels, table)
    return _mlp(emb, W1, b1, W2, b2, gamma, beta)
